# Initial kernel scaffold; baseline (speedup 1.0000x reference)
import jax
import jax.numpy as jnp
from jax.experimental import pallas as pl


def kernel(dst, src, index, weight):
    raise NotImplementedError("write your pallas kernel here")


# speedup vs baseline: 3.6565x; 3.6565x over previous
import functools

import jax
import jax.numpy as jnp
from jax import lax
from jax.experimental import pallas as pl
from jax.experimental.pallas import tpu as pltpu
from jax.experimental.pallas import tpu_sc as plsc

N_ROWS = 100000
D = 64
N_IDX = 524288

NC = 2
NS = 16
L = 16

NCHUNK = 10
OWN = N_ROWS // NCHUNK
INIT_PT = 632
EXT = NS * INIT_PT
TRASH = 1024
ACC_ROWS = EXT + TRASH

SHARE = N_IDX // NS
HALF = SHARE // 2
SUB = 2048
CAP = HALF + 2 * L
B = 128

WR_PT = 624
WR_REM = OWN - WR_PT * NS


def _body(dst_hbm, src_hbm, isrc_hbm, idst_hbm, w_hbm, out_hbm,
          acc, dstage, sstage, wstage, csrc, crel, cw, idxrow, relrow, rows):
    c = lax.axis_index("c")
    s = lax.axis_index("s")
    lanes = lax.iota(jnp.int32, L)
    one = jnp.full((L,), 1, jnp.int32)
    zero = jnp.full((L,), 0, jnp.int32)

    for p in range(NCHUNK // NC):
        cid = c * (NCHUNK // NC) + p
        lo = cid * OWN
        hi = lo + OWN
        start = jnp.minimum(lo, N_ROWS - EXT)
        woff = lo - start

        pltpu.sync_copy(dst_hbm.at[pl.ds(start + s * INIT_PT, INIT_PT)],
                        acc.at[pl.ds(s * INIT_PT, INIT_PT)])
        plsc.subcore_barrier()

        lov = jnp.full((L,), lo, jnp.int32)
        hiv = jnp.full((L,), hi, jnp.int32)
        startv = jnp.full((L,), start, jnp.int32)

        for h in range(2):
            half_base = s * SHARE + h * HALF

            def sub_step(j, n):
                base = half_base + j * SUB
                pltpu.sync_copy(idst_hbm.at[pl.ds(base, SUB)], dstage)
                pltpu.sync_copy(isrc_hbm.at[pl.ds(base, SUB)], sstage)
                pltpu.sync_copy(w_hbm.at[pl.ds(base, SUB)], wstage)

                def vec_step(k, n):
                    d = dstage[pl.ds(k * L, L)]
                    m = (d >= lov) & (d < hiv)
                    cum = plsc.cumsum(jnp.where(m, one, zero))
                    pos = (n + cum) - 1
                    plsc.store_scatter(csrc, [pos],
                                       sstage[pl.ds(k * L, L)], mask=m)
                    plsc.store_scatter(crel, [pos], d - startv, mask=m)
                    plsc.store_scatter(cw, [pos],
                                       wstage[pl.ds(k * L, L)], mask=m)
                    return n + cum[L - 1]

                return lax.fori_loop(0, SUB // L, vec_step, n)

            n = lax.fori_loop(0, HALF // SUB, sub_step, jnp.int32(0))

            for k in range(B // L):
                pos = n + k * L
                flat = pos + lanes
                csrc[pl.ds(pos, L)] = (flat * 37) & 32767
                crel[pl.ds(pos, L)] = EXT + (flat & (TRASH - 1))

            nb = (n + (B - 1)) // B

            def batch_step(b, carry):
                base = b * B
                for k in range(B // L):
                    idxrow[pl.ds(k * L, L)] = csrc[pl.ds(base + k * L, L)]
                    relrow[pl.ds(k * L, L)] = crel[pl.ds(base + k * L, L)]
                pltpu.sync_copy(src_hbm.at[idxrow], rows)

                def scale_group(g, carry):
                    wvec = cw[pl.ds(base + g * L, L)]
                    for i in range(L):
                        wv = wvec[i]
                        r = g * L + i
                        for k in range(D // L):
                            rows[r, pl.ds(k * L, L)] = (
                                rows[r, pl.ds(k * L, L)] * wv)
                    return carry

                lax.fori_loop(0, B // L, scale_group, 0)
                pltpu.sync_copy(rows, acc.at[relrow], add=True)
                return carry

            lax.fori_loop(0, nb, batch_step, 0)

        plsc.subcore_barrier()
        pltpu.sync_copy(acc.at[pl.ds(woff + s * WR_PT, WR_PT)],
                        out_hbm.at[pl.ds(lo + s * WR_PT, WR_PT)])

        @pl.when(s == 0)
        def _():
            pltpu.sync_copy(acc.at[pl.ds(woff + NS * WR_PT, WR_REM)],
                            out_hbm.at[pl.ds(lo + NS * WR_PT, WR_REM)])

        plsc.subcore_barrier()


@jax.jit
def kernel(dst, src, index, weight):
    mesh = plsc.VectorSubcoreMesh(core_axis_name="c", subcore_axis_name="s")
    run = pl.kernel(
        _body,
        out_type=jax.ShapeDtypeStruct((N_ROWS, D), jnp.float32),
        mesh=mesh,
        compiler_params=pltpu.CompilerParams(use_tc_tiling_on_sc=False,
                                             needs_layout_passes=False),
        scratch_types=[
            pltpu.VMEM_SHARED((ACC_ROWS, D), jnp.float32),
            pltpu.VMEM((SUB,), jnp.int32),
            pltpu.VMEM((SUB,), jnp.int32),
            pltpu.VMEM((SUB,), jnp.float32),
            pltpu.VMEM((CAP,), jnp.int32),
            pltpu.VMEM((CAP,), jnp.int32),
            pltpu.VMEM((CAP,), jnp.float32),
            pltpu.VMEM((B,), jnp.int32),
            pltpu.VMEM((B,), jnp.int32),
            pltpu.VMEM((B, D), jnp.float32),
        ],
    )
    return run(dst, src, index[0], index[1], weight[:, 0])


# speedup vs baseline: 5.2605x; 1.4387x over previous
import jax
import jax.numpy as jnp
from jax import lax
from jax.experimental import pallas as pl
from jax.experimental.pallas import tpu as pltpu
from jax.experimental.pallas import tpu_sc as plsc

N_ROWS = 100000
D = 64
N_IDX = 524288

NC = 2
NS = 16
L = 16

NCHUNK = 10
OWN = N_ROWS // NCHUNK
INIT_PT = 632
EXT = NS * INIT_PT
TRASH = 1024
ACC_ROWS = EXT + TRASH

SHARE = N_IDX // NS
HALF = SHARE // 2
SUB = 2048
NSUB = HALF // SUB
CAP = HALF + 2 * L
B = 128

WR_PT = 624
WR_REM = OWN - WR_PT * NS


def _body(dst_hbm, src_hbm, isrc_hbm, idst_hbm, w_hbm, out_hbm,
          acc, dstA, sstA, wstA, dstB, sstB, wstB, csrc, crel, cw,
          idxA, relA, idxB, relB, rowsA, rowsB,
          lsemA, lsemB, gsemA, gsemB, ssemA, ssemB):
    c = lax.axis_index("c")
    s = lax.axis_index("s")
    lanes = lax.iota(jnp.int32, L)
    one = jnp.full((L,), 1, jnp.int32)
    zero = jnp.full((L,), 0, jnp.int32)

    lslots = ((dstA, sstA, wstA, lsemA), (dstB, sstB, wstB, lsemB))

    def fire_loads(half_base, j, slot):
        dbuf, sbuf, wbuf, sem = slot
        base = half_base + j * SUB
        return (pltpu.async_copy(idst_hbm.at[pl.ds(base, SUB)], dbuf, sem),
                pltpu.async_copy(isrc_hbm.at[pl.ds(base, SUB)], sbuf, sem),
                pltpu.async_copy(w_hbm.at[pl.ds(base, SUB)], wbuf, sem))

    def pass_step(p, pcarry):
        cid = c * (NCHUNK // NC) + p
        lo = cid * OWN
        hi = lo + OWN
        start = jnp.minimum(lo, N_ROWS - EXT)
        woff = lo - start
        lov = jnp.full((L,), lo, jnp.int32)
        hiv = jnp.full((L,), hi, jnp.int32)
        startv = jnp.full((L,), start, jnp.int32)

        pltpu.sync_copy(dst_hbm.at[pl.ds(start + s * INIT_PT, INIT_PT)],
                        acc.at[pl.ds(s * INIT_PT, INIT_PT)])
        plsc.subcore_barrier()

        for h in range(2):
            half_base = s * SHARE + h * HALF

            def filter_sub(dbuf, sbuf, wbuf, n):
                def vec_step(k, n):
                    d = dbuf[pl.ds(k * L, L)]
                    m = (d >= lov) & (d < hiv)
                    cum = plsc.cumsum(jnp.where(m, one, zero))
                    pos = (n + cum) - 1
                    plsc.store_scatter(csrc, [pos],
                                       sbuf[pl.ds(k * L, L)], mask=m)
                    plsc.store_scatter(crel, [pos], d - startv, mask=m)
                    plsc.store_scatter(cw, [pos],
                                       wbuf[pl.ds(k * L, L)], mask=m)
                    return n + cum[L - 1]

                return lax.fori_loop(0, SUB // L, vec_step, n)

            n = jnp.int32(0)
            descs = [None, None]
            descs[0] = fire_loads(half_base, 0, lslots[0])
            for j in range(NSUB):
                sl = j & 1
                if j + 1 < NSUB:
                    descs[(j + 1) & 1] = fire_loads(half_base, j + 1,
                                                    lslots[(j + 1) & 1])
                for dd in descs[sl]:
                    dd.wait()
                dbuf, sbuf, wbuf, _ = lslots[sl]
                n = filter_sub(dbuf, sbuf, wbuf, n)

            for k in range(B // L):
                pos = n + k * L
                flat = pos + lanes
                csrc[pl.ds(pos, L)] = (flat * 37) & 32767
                crel[pl.ds(pos, L)] = EXT + (flat & (TRASH - 1))

            nb = (n + (B - 1)) // B

            def prep(b, idxr, relr):
                base = b * B
                for k in range(B // L):
                    idxr[pl.ds(k * L, L)] = csrc[pl.ds(base + k * L, L)]
                    relr[pl.ds(k * L, L)] = crel[pl.ds(base + k * L, L)]

            def scale(rows, b):
                base = b * B

                def scale_group(g, carry):
                    wvec = cw[pl.ds(base + g * L, L)]
                    for i in range(L):
                        wv = wvec[i]
                        r = g * L + i
                        for k in range(D // L):
                            rows[r, pl.ds(k * L, L)] = (
                                rows[r, pl.ds(k * L, L)] * wv)
                    return carry

                lax.fori_loop(0, B // L, scale_group, 0)

            def pair_step(q, carry):
                b0 = 2 * q
                b1 = b0 + 1
                prep(b0, idxA, relA)
                gA = pltpu.async_copy(src_hbm.at[idxA], rowsA, gsemA)

                @pl.when(b1 < nb)
                def _():
                    prep(b1, idxB, relB)
                    pltpu.async_copy(src_hbm.at[idxB], rowsB, gsemB)

                gA.wait()
                scale(rowsA, b0)
                sA = pltpu.async_copy(rowsA, acc.at[relA], ssemA, add=True)

                @pl.when(b1 < nb)
                def _():
                    pltpu.make_async_copy(src_hbm.at[idxB], rowsB,
                                          gsemB).wait()
                    scale(rowsB, b1)
                    pltpu.async_copy(rowsB, acc.at[relB], ssemB, add=True)

                sA.wait()

                @pl.when(b1 < nb)
                def _():
                    pltpu.make_async_copy(rowsB, acc.at[relB], ssemB).wait()

                return carry

            lax.fori_loop(0, (nb + 1) // 2, pair_step, 0)

        plsc.subcore_barrier()
        pltpu.sync_copy(acc.at[pl.ds(woff + s * WR_PT, WR_PT)],
                        out_hbm.at[pl.ds(lo + s * WR_PT, WR_PT)])

        @pl.when(s == 0)
        def _():
            pltpu.sync_copy(acc.at[pl.ds(woff + NS * WR_PT, WR_REM)],
                            out_hbm.at[pl.ds(lo + NS * WR_PT, WR_REM)])

        plsc.subcore_barrier()
        return pcarry

    lax.fori_loop(0, NCHUNK // NC, pass_step, 0)


@jax.jit
def kernel(dst, src, index, weight):
    mesh = plsc.VectorSubcoreMesh(core_axis_name="c", subcore_axis_name="s")
    run = pl.kernel(
        _body,
        out_type=jax.ShapeDtypeStruct((N_ROWS, D), jnp.float32),
        mesh=mesh,
        compiler_params=pltpu.CompilerParams(use_tc_tiling_on_sc=False,
                                             needs_layout_passes=False),
        scratch_types=[
            pltpu.VMEM_SHARED((ACC_ROWS, D), jnp.float32),
            pltpu.VMEM((SUB,), jnp.int32),
            pltpu.VMEM((SUB,), jnp.int32),
            pltpu.VMEM((SUB,), jnp.float32),
            pltpu.VMEM((SUB,), jnp.int32),
            pltpu.VMEM((SUB,), jnp.int32),
            pltpu.VMEM((SUB,), jnp.float32),
            pltpu.VMEM((CAP,), jnp.int32),
            pltpu.VMEM((CAP,), jnp.int32),
            pltpu.VMEM((CAP,), jnp.float32),
            pltpu.VMEM((B,), jnp.int32),
            pltpu.VMEM((B,), jnp.int32),
            pltpu.VMEM((B,), jnp.int32),
            pltpu.VMEM((B,), jnp.int32),
            pltpu.VMEM((B, D), jnp.float32),
            pltpu.VMEM((B, D), jnp.float32),
            pltpu.SemaphoreType.DMA,
            pltpu.SemaphoreType.DMA,
            pltpu.SemaphoreType.DMA,
            pltpu.SemaphoreType.DMA,
            pltpu.SemaphoreType.DMA,
            pltpu.SemaphoreType.DMA,
        ],
    )
    return run(dst, src, index[0], index[1], weight[:, 0])


# speedup vs baseline: 5.2635x; 1.0006x over previous
import jax
import jax.numpy as jnp
from jax import lax
from jax.experimental import pallas as pl
from jax.experimental.pallas import tpu as pltpu
from jax.experimental.pallas import tpu_sc as plsc

N_ROWS = 100000
D = 64
N_IDX = 524288

NC = 2
NS = 16
L = 16

NCHUNK = 10
OWN = N_ROWS // NCHUNK
INIT_PT = 632
EXT = NS * INIT_PT
TRASH = 1024
ACC_ROWS = EXT + TRASH

SHARE = N_IDX // NS
HALF = SHARE // 2
SUB = 2048
NSUB = HALF // SUB
CAP = HALF + 2 * L
B = 128

WR_PT = 624
WR_REM = OWN - WR_PT * NS


def _body(dst_hbm, src_hbm, isrc_hbm, idst_hbm, w_hbm, out_hbm,
          acc, dstA, sstA, wstA, dstB, sstB, wstB, cpk, cw,
          idxA, relA, idxB, relB, rowsA, rowsB,
          lsemA, lsemB, gsemA, gsemB, ssemA, ssemB):
    c = lax.axis_index("c")
    s = lax.axis_index("s")
    lanes = lax.iota(jnp.int32, L)
    one = jnp.full((L,), 1, jnp.int32)
    zero = jnp.full((L,), 0, jnp.int32)
    ownv = jnp.full((L,), OWN, jnp.uint32)

    lslots = ((dstA, sstA, wstA, lsemA), (dstB, sstB, wstB, lsemB))

    def fire_loads(half_base, j, slot):
        dbuf, sbuf, wbuf, sem = slot
        base = half_base + j * SUB
        return (pltpu.async_copy(idst_hbm.at[pl.ds(base, SUB)], dbuf, sem),
                pltpu.async_copy(isrc_hbm.at[pl.ds(base, SUB)], sbuf, sem),
                pltpu.async_copy(w_hbm.at[pl.ds(base, SUB)], wbuf, sem))

    def pass_step(p, pcarry):
        cid = c * (NCHUNK // NC) + p
        lo = cid * OWN
        hi = lo + OWN
        start = jnp.minimum(lo, N_ROWS - EXT)
        woff = lo - start
        lov = jnp.full((L,), lo, jnp.int32)
        startv = jnp.full((L,), start, jnp.int32)

        pltpu.sync_copy(dst_hbm.at[pl.ds(start + s * INIT_PT, INIT_PT)],
                        acc.at[pl.ds(s * INIT_PT, INIT_PT)])
        plsc.subcore_barrier()

        for h in range(2):
            half_base = s * SHARE + h * HALF

            def filter_sub(dbuf, sbuf, wbuf, n):
                def vec_step(k, n):
                    d = dbuf[pl.ds(k * L, L)]
                    m = (d - lov).astype(jnp.uint32) < ownv
                    cum = plsc.cumsum(jnp.where(m, one, zero))
                    pos = (n + cum) - 1
                    packed = sbuf[pl.ds(k * L, L)] * 16384 + (d - startv)
                    plsc.store_scatter(cpk, [pos], packed, mask=m)
                    plsc.store_scatter(cw, [pos],
                                       wbuf[pl.ds(k * L, L)], mask=m)
                    return n + cum[L - 1]

                return lax.fori_loop(0, SUB // L, vec_step, n)

            n = jnp.int32(0)
            descs = [None, None]
            descs[0] = fire_loads(half_base, 0, lslots[0])
            for j in range(NSUB):
                sl = j & 1
                if j + 1 < NSUB:
                    descs[(j + 1) & 1] = fire_loads(half_base, j + 1,
                                                    lslots[(j + 1) & 1])
                for dd in descs[sl]:
                    dd.wait()
                dbuf, sbuf, wbuf, _ = lslots[sl]
                n = filter_sub(dbuf, sbuf, wbuf, n)

            for k in range(B // L):
                pos = n + k * L
                flat = pos + lanes
                cpk[pl.ds(pos, L)] = (((flat * 37) & 32767) * 16384
                                      + (EXT + (flat & (TRASH - 1))))

            nb = (n + (B - 1)) // B

            def prep(b, idxr, relr):
                base = b * B
                for k in range(B // L):
                    pk = cpk[pl.ds(base + k * L, L)]
                    idxr[pl.ds(k * L, L)] = pk >> 14
                    relr[pl.ds(k * L, L)] = pk & 16383

            def scale(rows, b):
                base = b * B

                def scale_group(g, carry):
                    wvec = cw[pl.ds(base + g * L, L)]
                    for i in range(L):
                        wv = wvec[i]
                        r = g * L + i
                        for k in range(D // L):
                            rows[r, pl.ds(k * L, L)] = (
                                rows[r, pl.ds(k * L, L)] * wv)
                    return carry

                lax.fori_loop(0, B // L, scale_group, 0)

            def pair_step(q, carry):
                b0 = 2 * q
                b1 = b0 + 1
                prep(b0, idxA, relA)
                gA = pltpu.async_copy(src_hbm.at[idxA], rowsA, gsemA)

                @pl.when(b1 < nb)
                def _():
                    prep(b1, idxB, relB)
                    pltpu.async_copy(src_hbm.at[idxB], rowsB, gsemB)

                gA.wait()
                scale(rowsA, b0)
                sA = pltpu.async_copy(rowsA, acc.at[relA], ssemA, add=True)

                @pl.when(b1 < nb)
                def _():
                    pltpu.make_async_copy(src_hbm.at[idxB], rowsB,
                                          gsemB).wait()
                    scale(rowsB, b1)
                    pltpu.async_copy(rowsB, acc.at[relB], ssemB, add=True)

                sA.wait()

                @pl.when(b1 < nb)
                def _():
                    pltpu.make_async_copy(rowsB, acc.at[relB], ssemB).wait()

                return carry

            lax.fori_loop(0, (nb + 1) // 2, pair_step, 0)

        plsc.subcore_barrier()
        pltpu.sync_copy(acc.at[pl.ds(woff + s * WR_PT, WR_PT)],
                        out_hbm.at[pl.ds(lo + s * WR_PT, WR_PT)])

        @pl.when(s == 0)
        def _():
            pltpu.sync_copy(acc.at[pl.ds(woff + NS * WR_PT, WR_REM)],
                            out_hbm.at[pl.ds(lo + NS * WR_PT, WR_REM)])

        plsc.subcore_barrier()
        return pcarry

    lax.fori_loop(0, NCHUNK // NC, pass_step, 0)


@jax.jit
def kernel(dst, src, index, weight):
    mesh = plsc.VectorSubcoreMesh(core_axis_name="c", subcore_axis_name="s")
    run = pl.kernel(
        _body,
        out_type=jax.ShapeDtypeStruct((N_ROWS, D), jnp.float32),
        mesh=mesh,
        compiler_params=pltpu.CompilerParams(use_tc_tiling_on_sc=False,
                                             needs_layout_passes=False),
        scratch_types=[
            pltpu.VMEM_SHARED((ACC_ROWS, D), jnp.float32),
            pltpu.VMEM((SUB,), jnp.int32),
            pltpu.VMEM((SUB,), jnp.int32),
            pltpu.VMEM((SUB,), jnp.float32),
            pltpu.VMEM((SUB,), jnp.int32),
            pltpu.VMEM((SUB,), jnp.int32),
            pltpu.VMEM((SUB,), jnp.float32),
            pltpu.VMEM((CAP,), jnp.int32),
            pltpu.VMEM((CAP,), jnp.float32),
            pltpu.VMEM((B,), jnp.int32),
            pltpu.VMEM((B,), jnp.int32),
            pltpu.VMEM((B,), jnp.int32),
            pltpu.VMEM((B,), jnp.int32),
            pltpu.VMEM((B, D), jnp.float32),
            pltpu.VMEM((B, D), jnp.float32),
            pltpu.SemaphoreType.DMA,
            pltpu.SemaphoreType.DMA,
            pltpu.SemaphoreType.DMA,
            pltpu.SemaphoreType.DMA,
            pltpu.SemaphoreType.DMA,
            pltpu.SemaphoreType.DMA,
        ],
    )
    return run(dst, src, index[0], index[1], weight[:, 0])


# speedup vs baseline: 5.8868x; 1.1184x over previous
import jax
import jax.numpy as jnp
from jax import lax
from jax.experimental import pallas as pl
from jax.experimental.pallas import tpu as pltpu
from jax.experimental.pallas import tpu_sc as plsc

N_ROWS = 100000
D = 64
N_IDX = 524288

NC = 2
NS = 16
L = 16

NCHUNK = 10
OWN = N_ROWS // NCHUNK
INIT_PT = 632
EXT = NS * INIT_PT
TRASH = 1024
ACC_ROWS = EXT + TRASH

SHARE = N_IDX // NS
HALF = SHARE // 2
SUB = 2048
NSUB = HALF // SUB
CAP = HALF + 2 * L
B = 128

WR_PT = 624
WR_REM = OWN - WR_PT * NS


def _body(dst_hbm, src_hbm, isrc_hbm, idst_hbm, w_hbm, out_hbm,
          acc, dstA, sstA, wstA, dstB, sstB, wstB, cpk, cw,
          idxA, relA, idxB, relB, idxC, relC, rowsA, rowsB, rowsC,
          lsemA, lsemB, gsemA, gsemB, gsemC, ssemA, ssemB, ssemC):
    c = lax.axis_index("c")
    s = lax.axis_index("s")
    lanes = lax.iota(jnp.int32, L)
    one = jnp.full((L,), 1, jnp.int32)
    zero = jnp.full((L,), 0, jnp.int32)
    ownv = jnp.full((L,), OWN, jnp.uint32)

    lslots = ((dstA, sstA, wstA, lsemA), (dstB, sstB, wstB, lsemB))

    def fire_loads(half_base, j, slot):
        dbuf, sbuf, wbuf, sem = slot
        base = half_base + j * SUB
        return (pltpu.async_copy(idst_hbm.at[pl.ds(base, SUB)], dbuf, sem),
                pltpu.async_copy(isrc_hbm.at[pl.ds(base, SUB)], sbuf, sem),
                pltpu.async_copy(w_hbm.at[pl.ds(base, SUB)], wbuf, sem))

    def pass_step(p, pcarry):
        cid = c * (NCHUNK // NC) + p
        lo = cid * OWN
        hi = lo + OWN
        start = jnp.minimum(lo, N_ROWS - EXT)
        woff = lo - start
        lov = jnp.full((L,), lo, jnp.int32)
        startv = jnp.full((L,), start, jnp.int32)

        pltpu.sync_copy(dst_hbm.at[pl.ds(start + s * INIT_PT, INIT_PT)],
                        acc.at[pl.ds(s * INIT_PT, INIT_PT)])
        plsc.subcore_barrier()

        for h in range(2):
            half_base = s * SHARE + h * HALF

            def filter_sub(dbuf, sbuf, wbuf, n):
                def vec_step(k, n):
                    d = dbuf[pl.ds(k * L, L)]
                    m = (d - lov).astype(jnp.uint32) < ownv
                    cum = plsc.cumsum(jnp.where(m, one, zero))
                    pos = (n + cum) - 1
                    packed = sbuf[pl.ds(k * L, L)] * 16384 + (d - startv)
                    plsc.store_scatter(cpk, [pos], packed, mask=m)
                    plsc.store_scatter(cw, [pos],
                                       wbuf[pl.ds(k * L, L)], mask=m)
                    return n + cum[L - 1]

                return lax.fori_loop(0, SUB // L, vec_step, n)

            n = jnp.int32(0)
            descs = [None, None]
            descs[0] = fire_loads(half_base, 0, lslots[0])
            for j in range(NSUB):
                sl = j & 1
                if j + 1 < NSUB:
                    descs[(j + 1) & 1] = fire_loads(half_base, j + 1,
                                                    lslots[(j + 1) & 1])
                for dd in descs[sl]:
                    dd.wait()
                dbuf, sbuf, wbuf, _ = lslots[sl]
                n = filter_sub(dbuf, sbuf, wbuf, n)

            for k in range(B // L):
                pos = n + k * L
                flat = pos + lanes
                cpk[pl.ds(pos, L)] = (((flat * 37) & 32767) * 16384
                                      + (EXT + (flat & (TRASH - 1))))

            nb = (n + (B - 1)) // B

            def prep(b, idxr, relr):
                base = b * B
                for k in range(B // L):
                    pk = cpk[pl.ds(base + k * L, L)]
                    idxr[pl.ds(k * L, L)] = pk >> 14
                    relr[pl.ds(k * L, L)] = pk & 16383

            def scale(rows, b):
                base = b * B

                def scale_group(g, carry):
                    wvec = cw[pl.ds(base + g * L, L)]
                    for i in range(L):
                        wv = wvec[i]
                        r = g * L + i
                        for k in range(D // L):
                            rows[r, pl.ds(k * L, L)] = (
                                rows[r, pl.ds(k * L, L)] * wv)
                    return carry

                lax.fori_loop(0, B // L, scale_group, 0)

            slots = ((idxA, relA, rowsA, gsemA, ssemA),
                     (idxB, relB, rowsB, gsemB, ssemB),
                     (idxC, relC, rowsC, gsemC, ssemC))

            def wait_gather(slot):
                idxr, _, rows, gsem, _ = slot
                pltpu.make_async_copy(src_hbm.at[idxr], rows, gsem).wait()

            def wait_scatter(slot):
                _, relr, rows, _, ssem = slot
                pltpu.make_async_copy(rows, acc.at[relr], ssem).wait()

            def launch(b, slot):
                idxr, relr, rows, gsem, _ = slot
                prep(b, idxr, relr)
                pltpu.async_copy(src_hbm.at[idxr], rows, gsem)

            def finish(b, slot):
                _, relr, rows, _, ssem = slot
                wait_gather(slot)
                scale(rows, b)
                pltpu.async_copy(rows, acc.at[relr], ssem, add=True)

            @pl.when(0 < nb)
            def _():
                launch(0, slots[0])

            @pl.when(1 < nb)
            def _():
                launch(1, slots[1])

            def tri_step(q, carry):
                b0 = 3 * q

                finish(b0, slots[0])

                @pl.when(b0 + 2 < nb)
                def _():
                    @pl.when(q > 0)
                    def _():
                        wait_scatter(slots[2])
                    launch(b0 + 2, slots[2])

                @pl.when(b0 + 1 < nb)
                def _():
                    finish(b0 + 1, slots[1])

                @pl.when(b0 + 3 < nb)
                def _():
                    wait_scatter(slots[0])
                    launch(b0 + 3, slots[0])

                @pl.when(b0 + 2 < nb)
                def _():
                    finish(b0 + 2, slots[2])

                @pl.when(b0 + 4 < nb)
                def _():
                    wait_scatter(slots[1])
                    launch(b0 + 4, slots[1])

                return carry

            lax.fori_loop(0, (nb + 2) // 3, tri_step, 0)

            for i in range(3):
                @pl.when(i < nb)
                def _(i=i):
                    wait_scatter(slots[i])

        plsc.subcore_barrier()
        pltpu.sync_copy(acc.at[pl.ds(woff + s * WR_PT, WR_PT)],
                        out_hbm.at[pl.ds(lo + s * WR_PT, WR_PT)])

        @pl.when(s == 0)
        def _():
            pltpu.sync_copy(acc.at[pl.ds(woff + NS * WR_PT, WR_REM)],
                            out_hbm.at[pl.ds(lo + NS * WR_PT, WR_REM)])

        plsc.subcore_barrier()
        return pcarry

    lax.fori_loop(0, NCHUNK // NC, pass_step, 0)


@jax.jit
def kernel(dst, src, index, weight):
    mesh = plsc.VectorSubcoreMesh(core_axis_name="c", subcore_axis_name="s")
    run = pl.kernel(
        _body,
        out_type=jax.ShapeDtypeStruct((N_ROWS, D), jnp.float32),
        mesh=mesh,
        compiler_params=pltpu.CompilerParams(use_tc_tiling_on_sc=False,
                                             needs_layout_passes=False),
        scratch_types=[
            pltpu.VMEM_SHARED((ACC_ROWS, D), jnp.float32),
            pltpu.VMEM((SUB,), jnp.int32),
            pltpu.VMEM((SUB,), jnp.int32),
            pltpu.VMEM((SUB,), jnp.float32),
            pltpu.VMEM((SUB,), jnp.int32),
            pltpu.VMEM((SUB,), jnp.int32),
            pltpu.VMEM((SUB,), jnp.float32),
            pltpu.VMEM((CAP,), jnp.int32),
            pltpu.VMEM((CAP,), jnp.float32),
            pltpu.VMEM((B,), jnp.int32),
            pltpu.VMEM((B,), jnp.int32),
            pltpu.VMEM((B,), jnp.int32),
            pltpu.VMEM((B,), jnp.int32),
            pltpu.VMEM((B,), jnp.int32),
            pltpu.VMEM((B,), jnp.int32),
            pltpu.VMEM((B, D), jnp.float32),
            pltpu.VMEM((B, D), jnp.float32),
            pltpu.VMEM((B, D), jnp.float32),
            pltpu.SemaphoreType.DMA,
            pltpu.SemaphoreType.DMA,
            pltpu.SemaphoreType.DMA,
            pltpu.SemaphoreType.DMA,
            pltpu.SemaphoreType.DMA,
            pltpu.SemaphoreType.DMA,
            pltpu.SemaphoreType.DMA,
            pltpu.SemaphoreType.DMA,
        ],
    )
    return run(dst, src, index[0], index[1], weight[:, 0])
